# baseline (speedup 1.0000x reference)
import functools

import jax
import jax.numpy as jnp
from jax import lax
from jax.experimental import pallas as pl
from jax.experimental.pallas import tpu as pltpu
from jax.experimental.pallas import tpu_sc as plsc

VOCAB = 100000
HIDDEN = 128
BATCH = 1024


_NC, _NS = 1, 16
_NW = _NC * _NS
_B_PER_W = BATCH // _NW

@functools.cache
def _make_sc_gather():
    mesh = plsc.VectorSubcoreMesh(
        core_axis_name="c", subcore_axis_name="s", num_cores=_NC, num_subcores=_NS
    )

    n_chunks = 4
    chunk = _B_PER_W // n_chunks

    @functools.partial(
        pl.kernel,
        mesh=mesh,
        out_type=jax.ShapeDtypeStruct((BATCH, HIDDEN), jnp.float32),
        scratch_types=[
            pltpu.VMEM((_B_PER_W,), jnp.int32),
            *[pltpu.VMEM((chunk, HIDDEN), jnp.float32) for _ in range(n_chunks)],
            *[pltpu.SemaphoreType.DMA for _ in range(n_chunks)],
            *[pltpu.SemaphoreType.DMA for _ in range(n_chunks)],
        ],
    )
    def _sc_gather(table_hbm, idx_hbm, out_hbm, idx_v, *scratch):
        rows = scratch[:n_chunks]
        gsem = scratch[n_chunks:2 * n_chunks]
        wsem = scratch[2 * n_chunks:]
        wid = lax.axis_index("s") * _NC + lax.axis_index("c")
        base = wid * _B_PER_W
        pltpu.sync_copy(idx_hbm.at[pl.ds(base, _B_PER_W)], idx_v)
        gathers = []
        for k in range(n_chunks):
            gathers.append(pltpu.async_copy(
                table_hbm.at[idx_v.at[pl.ds(k * chunk, chunk)]],
                rows[k], gsem[k]))
        writes = []
        for k in range(n_chunks):
            gathers[k].wait()
            writes.append(pltpu.async_copy(
                rows[k], out_hbm.at[pl.ds(base + k * chunk, chunk)], wsem[k]))
        for w in writes:
            w.wait()

    return _sc_gather


_TILE_V = 5120
_GRID_V = pl.cdiv(VOCAB, _TILE_V)


def _matmul_t_body(x_ref, w_ref, b_ref, o_ref):
    acc = jax.lax.dot_general(
        w_ref[...], x_ref[...],
        dimension_numbers=(((1,), (1,)), ((), ())),
        preferred_element_type=jnp.float32,
    )
    o_ref[...] = acc + b_ref[...][:, None]


def _tc_head(x, head_w, head_b):
    w_t = head_w.T
    out_t = pl.pallas_call(
        _matmul_t_body,
        grid=(_GRID_V,),
        in_specs=[
            pl.BlockSpec((BATCH, HIDDEN), lambda j: (0, 0)),
            pl.BlockSpec((_TILE_V, HIDDEN), lambda j: (j, 0)),
            pl.BlockSpec((_TILE_V,), lambda j: (j,)),
        ],
        out_specs=pl.BlockSpec((_TILE_V, BATCH), lambda j: (j, 0)),
        out_shape=jax.ShapeDtypeStruct((VOCAB, BATCH), jnp.float32),
        compiler_params=pltpu.CompilerParams(
            dimension_semantics=("arbitrary",),
        ),
    )(x, w_t, head_b)
    return out_t.T


def kernel(input_ids, embed_table, head_w, head_b):
    idx = input_ids.astype(jnp.int32)
    x = _make_sc_gather()(embed_table, idx)
    return _tc_head(x, head_w, head_b)


# speedup vs baseline: 1.0022x; 1.0022x over previous
import functools

import jax
import jax.numpy as jnp
from jax import lax
from jax.experimental import pallas as pl
from jax.experimental.pallas import tpu as pltpu
from jax.experimental.pallas import tpu_sc as plsc

VOCAB = 100000
HIDDEN = 128
BATCH = 1024


_NC, _NS = 1, 16
_NW = _NC * _NS
_B_PER_W = BATCH // _NW

@functools.cache
def _make_sc_gather():
    mesh = plsc.VectorSubcoreMesh(
        core_axis_name="c", subcore_axis_name="s", num_cores=_NC, num_subcores=_NS
    )

    n_chunks = 2
    chunk = _B_PER_W // n_chunks

    @functools.partial(
        pl.kernel,
        mesh=mesh,
        out_type=jax.ShapeDtypeStruct((BATCH, HIDDEN), jnp.float32),
        scratch_types=[
            pltpu.VMEM((_B_PER_W,), jnp.int32),
            *[pltpu.VMEM((chunk, HIDDEN), jnp.float32) for _ in range(n_chunks)],
            *[pltpu.SemaphoreType.DMA for _ in range(n_chunks)],
            *[pltpu.SemaphoreType.DMA for _ in range(n_chunks)],
        ],
    )
    def _sc_gather(table_hbm, idx_hbm, out_hbm, idx_v, *scratch):
        rows = scratch[:n_chunks]
        gsem = scratch[n_chunks:2 * n_chunks]
        wsem = scratch[2 * n_chunks:]
        wid = lax.axis_index("s") * _NC + lax.axis_index("c")
        base = wid * _B_PER_W
        pltpu.sync_copy(idx_hbm.at[pl.ds(base, _B_PER_W)], idx_v)
        gathers = []
        for k in range(n_chunks):
            gathers.append(pltpu.async_copy(
                table_hbm.at[idx_v.at[pl.ds(k * chunk, chunk)]],
                rows[k], gsem[k]))
        writes = []
        for k in range(n_chunks):
            gathers[k].wait()
            writes.append(pltpu.async_copy(
                rows[k], out_hbm.at[pl.ds(base + k * chunk, chunk)], wsem[k]))
        for w in writes:
            w.wait()

    return _sc_gather


_TILE_V = 5120
_GRID_V = pl.cdiv(VOCAB, _TILE_V)


def _matmul_t_body(x_ref, w_ref, b_ref, o_ref):
    acc = jax.lax.dot_general(
        w_ref[...], x_ref[...],
        dimension_numbers=(((1,), (1,)), ((), ())),
        preferred_element_type=jnp.float32,
    )
    o_ref[...] = acc + b_ref[...][:, None]


def _tc_head(x, head_w, head_b):
    w_t = head_w.T
    out_t = pl.pallas_call(
        _matmul_t_body,
        grid=(_GRID_V,),
        in_specs=[
            pl.BlockSpec((BATCH, HIDDEN), lambda j: (0, 0)),
            pl.BlockSpec((_TILE_V, HIDDEN), lambda j: (j, 0)),
            pl.BlockSpec((_TILE_V,), lambda j: (j,)),
        ],
        out_specs=pl.BlockSpec((_TILE_V, BATCH), lambda j: (j, 0)),
        out_shape=jax.ShapeDtypeStruct((VOCAB, BATCH), jnp.float32),
        compiler_params=pltpu.CompilerParams(
            dimension_semantics=("arbitrary",),
        ),
    )(x, w_t, head_b)
    return out_t.T


def kernel(input_ids, embed_table, head_w, head_b):
    idx = input_ids.astype(jnp.int32)
    x = _make_sc_gather()(embed_table, idx)
    return _tc_head(x, head_w, head_b)
